# baseline (speedup 1.0000x reference)
import functools

import jax
import jax.numpy as jnp
from jax import lax
from jax.experimental import pallas as pl
from jax.experimental.pallas import tpu as pltpu
from jax.experimental.pallas import tpu_sc as plsc

K_NB = 16
_BIG = 1e9
_L = 16
_CHK = 64


def _prep_kernel(x_ref, xnt_ref, d2_ref):
    x = x_ref[0]
    xt = jnp.transpose(x)
    nrm = jnp.sqrt(jnp.sum(xt * xt, axis=1, keepdims=True))
    xn = xt * (1.0 / jnp.maximum(nrm, 1e-12))
    sq = jnp.sum(xn * xn, axis=1, keepdims=True)
    g = lax.dot_general(xn, xn, (((1,), (1,)), ((), ())),
                        preferred_element_type=jnp.float32)
    d2 = sq + jnp.transpose(sq) - 2.0 * g
    d2 = jnp.maximum(d2, 0.0)
    N = d2.shape[0]
    rowid = lax.broadcasted_iota(jnp.int32, (N, N), 0)
    colid = lax.broadcasted_iota(jnp.int32, (N, N), 1)
    d2 = jnp.where(rowid == colid, _BIG, d2)
    xnt_ref[0] = xn.astype(jnp.bfloat16)
    d2_ref[0] = d2


def _knn_sc_kernel(xnt_hbm, d2_hbm, md_hbm, xnt_v, d2_v, md_v, *, n, c):
    nc = 2
    b = lax.axis_index("s") * nc + lax.axis_index("c")
    pltpu.sync_copy(xnt_hbm.at[b], xnt_v)
    pltpu.sync_copy(d2_hbm.at[b], d2_v)
    iot = lax.iota(jnp.int32, _L)
    big16 = jnp.full((_L,), _BIG, jnp.float32)
    nslc = n // _L
    ncb = c // 32
    perms = [(iot + sh) % _L for sh in (8, 4, 2, 1)]

    gdn = lax.GatherDimensionNumbers(
        offset_dims=(), collapsed_slice_dims=(0,), start_index_map=(0,))

    def lane_min(v):
        for p in perms:
            rolled = lax.gather(
                v, p[:, None], gdn, slice_sizes=(1,),
                mode=lax.GatherScatterMode.PROMISE_IN_BOUNDS)
            v = jnp.minimum(v, rolled)
        return v

    def chunk_body(ch, carry0):
        def row_body(il, carry1):
            i = ch * _CHK + il
            xi = [xnt_v[i, cc] for cc in range(ncb)]

            def k_body(_, md):
                mv = big16
                mi = iot
                for j in range(nslc):
                    v = d2_v[i, j]
                    idxv = iot + j * _L
                    lt = v < mv
                    mv = jnp.where(lt, v, mv)
                    mi = jnp.where(lt, idxv, mi)
                m = lane_min(mv)
                sel = jnp.where(mv == m, mi, n + n)
                jv = lane_min(sel)
                jsc = jv[0]
                jhi = jsc // _L
                jlo = jsc % _L
                vslc = d2_v[i, jhi]
                d2_v[i, jhi] = jnp.where(iot == jlo, _BIG, vslc)
                out = []
                for cc in range(ncb):
                    nb = xnt_v[jsc, cc]
                    out.append(jnp.maximum(md[cc], jnp.abs(xi[cc] - nb)))
                return tuple(out)

            md0 = tuple(
                jnp.zeros((32,), jnp.bfloat16) for _ in range(ncb))
            mdf = lax.fori_loop(0, K_NB, k_body, md0)
            for cc in range(ncb):
                md_v[il, cc] = mdf[cc]
            return carry1

        lax.fori_loop(0, _CHK, row_body, 0)
        pltpu.sync_copy(md_v, md_hbm.at[b, pl.ds(ch * _CHK, _CHK)])
        return carry0

    lax.fori_loop(0, n // _CHK, chunk_body, 0)


def _conv_kernel(we_ref, wo_ref, bias_ref, xn_ref, md_ref,
                 y_ref, s1_ref, s2_ref):
    b = pl.program_id(0)
    y = lax.dot_general(we_ref[...], xn_ref[0], (((1,), (1,)), ((), ())),
                        preferred_element_type=jnp.float32)
    y = y + lax.dot_general(wo_ref[...], md_ref[0], (((1,), (1,)), ((), ())),
                            preferred_element_type=jnp.float32)
    y = y + bias_ref[...]
    y_ref[0] = y
    ps1 = jnp.sum(y, axis=1, keepdims=True)
    ps2 = jnp.sum(y * y, axis=1, keepdims=True)

    @pl.when(b == 0)
    def _():
        s1_ref[...] = ps1
        s2_ref[...] = ps2

    @pl.when(b != 0)
    def _():
        s1_ref[...] += ps1
        s2_ref[...] += ps2


def _bn_gelu_kernel(y_ref, s1_ref, s2_ref, gamma_ref, beta_ref, o_ref,
                    *, count):
    mean = s1_ref[...] * (1.0 / count)
    var = s2_ref[...] * (1.0 / count) - mean * mean
    scale = gamma_ref[...] * lax.rsqrt(var + 1e-5)
    shift = beta_ref[...] - mean * scale
    yn = y_ref[0] * scale + shift
    o_ref[0] = yn * 0.5 * (1.0 + lax.erf(yn * 0.7071067811865476))


def kernel(x, W, b, gamma, beta):
    B, C, N = x.shape
    O = W.shape[0]
    we = W[:, 0::2].astype(jnp.bfloat16)
    wo = W[:, 1::2].astype(jnp.bfloat16)

    xnt, d2 = pl.pallas_call(
        _prep_kernel,
        grid=(B,),
        in_specs=[pl.BlockSpec((1, C, N), lambda i: (i, 0, 0))],
        out_specs=[pl.BlockSpec((1, N, C), lambda i: (i, 0, 0)),
                   pl.BlockSpec((1, N, N), lambda i: (i, 0, 0))],
        out_shape=[jax.ShapeDtypeStruct((B, N, C), jnp.bfloat16),
                   jax.ShapeDtypeStruct((B, N, N), jnp.float32)],
    )(x)

    md4 = pl.kernel(
        functools.partial(_knn_sc_kernel, n=N, c=C),
        mesh=plsc.VectorSubcoreMesh(core_axis_name="c", subcore_axis_name="s"),
        compiler_params=pltpu.CompilerParams(use_tc_tiling_on_sc=False),
        out_type=jax.ShapeDtypeStruct((B, N, C // 32, 32), jnp.bfloat16),
        scratch_types=[
            pltpu.VMEM((N, C // 32, 32), jnp.bfloat16),
            pltpu.VMEM((N, N // _L, _L), jnp.float32),
            pltpu.VMEM((_CHK, C // 32, 32), jnp.bfloat16),
        ],
    )(xnt.reshape(B, N, C // 32, 32), d2.reshape(B, N, N // _L, _L))
    md = md4.reshape(B, N, C)

    y, s1, s2 = pl.pallas_call(
        _conv_kernel,
        grid=(B,),
        in_specs=[pl.BlockSpec((O, C), lambda i: (0, 0)),
                  pl.BlockSpec((O, C), lambda i: (0, 0)),
                  pl.BlockSpec((O, 1), lambda i: (0, 0)),
                  pl.BlockSpec((1, N, C), lambda i: (i, 0, 0)),
                  pl.BlockSpec((1, N, C), lambda i: (i, 0, 0))],
        out_specs=[pl.BlockSpec((1, O, N), lambda i: (i, 0, 0)),
                   pl.BlockSpec((O, 1), lambda i: (0, 0)),
                   pl.BlockSpec((O, 1), lambda i: (0, 0))],
        out_shape=[jax.ShapeDtypeStruct((B, O, N), jnp.float32),
                   jax.ShapeDtypeStruct((O, 1), jnp.float32),
                   jax.ShapeDtypeStruct((O, 1), jnp.float32)],
    )(we, wo, b.reshape(O, 1), xnt, md)

    out = pl.pallas_call(
        functools.partial(_bn_gelu_kernel, count=float(B * N)),
        grid=(B,),
        in_specs=[pl.BlockSpec((1, O, N), lambda i: (i, 0, 0)),
                  pl.BlockSpec((O, 1), lambda i: (0, 0)),
                  pl.BlockSpec((O, 1), lambda i: (0, 0)),
                  pl.BlockSpec((O, 1), lambda i: (0, 0)),
                  pl.BlockSpec((O, 1), lambda i: (0, 0))],
        out_specs=pl.BlockSpec((1, O, N), lambda i: (i, 0, 0)),
        out_shape=jax.ShapeDtypeStruct((B, O, N), jnp.float32),
    )(y, s1, s2, gamma.reshape(O, 1), beta.reshape(O, 1))

    return out.reshape(B, O, N, 1)


# speedup vs baseline: 1.6056x; 1.6056x over previous
import functools

import jax
import jax.numpy as jnp
from jax import lax
from jax.experimental import pallas as pl
from jax.experimental.pallas import tpu as pltpu
from jax.experimental.pallas import tpu_sc as plsc

K_NB = 16
_BIG = 1e9
_L = 16
_CHK = 64


def _prep_kernel(x_ref, xnt_ref, nn_ref):
    x = x_ref[0]
    xt = jnp.transpose(x)
    nrm = jnp.sqrt(jnp.sum(xt * xt, axis=1, keepdims=True))
    xn = xt * (1.0 / jnp.maximum(nrm, 1e-12))
    sq = jnp.sum(xn * xn, axis=1, keepdims=True)
    g = lax.dot_general(xn, xn, (((1,), (1,)), ((), ())),
                        preferred_element_type=jnp.float32)
    d2 = sq + jnp.transpose(sq) - 2.0 * g
    d2 = jnp.maximum(d2, 0.0)
    N = d2.shape[0]
    rowid = lax.broadcasted_iota(jnp.int32, (N, N), 0)
    colid = lax.broadcasted_iota(jnp.int32, (N, N), 1)
    d2 = jnp.where(rowid == colid, _BIG, d2)
    cols = []
    for _ in range(K_NB):
        m = jnp.min(d2, axis=1, keepdims=True)
        cand = jnp.where(d2 == m, colid, N)
        first = jnp.min(cand, axis=1, keepdims=True)
        cols.append(first)
        d2 = jnp.where(colid == first, _BIG, d2)
    xnt_ref[0] = xn
    nn_ref[0] = jnp.concatenate(cols, axis=1)


def _knn_sc_kernel(xnt_hbm, nn_hbm, md_hbm, xnt_v, nn_v, md_v, *, n, c):
    nc = 2
    b = lax.axis_index("s") * nc + lax.axis_index("c")
    pltpu.sync_copy(xnt_hbm.at[b], xnt_v)
    pltpu.sync_copy(nn_hbm.at[b], nn_v)
    iot = lax.iota(jnp.int32, _L)
    ncc = c // _L
    nh = ncc // 2

    def chunk_body(ch, carry0):
        def row_body(il, carry1):
            i = ch * _CHK + il
            idxv = nn_v[i, :]
            for h in range(2):
                lo = h * nh
                mx = [xnt_v[i, pl.ds((lo + cc) * _L, _L)] for cc in range(nh)]
                mn = list(mx)
                for t in range(K_NB):
                    jsc = idxv[t]
                    for cc in range(nh):
                        nb = xnt_v[jsc, pl.ds((lo + cc) * _L, _L)]
                        mx[cc] = jnp.maximum(mx[cc], nb)
                        mn[cc] = jnp.minimum(mn[cc], nb)
                for cc in range(nh):
                    xi = xnt_v[i, pl.ds((lo + cc) * _L, _L)]
                    md_v[il, pl.ds((lo + cc) * _L, _L)] = jnp.maximum(
                        mx[cc] - xi, xi - mn[cc])
            return carry1

        lax.fori_loop(0, _CHK, row_body, 0)
        pltpu.sync_copy(md_v, md_hbm.at[b, pl.ds(ch * _CHK, _CHK)])
        return carry0

    lax.fori_loop(0, n // _CHK, chunk_body, 0)


def _conv_kernel(we_ref, wo_ref, bias_ref, xn_ref, md_ref,
                 y_ref, s1_ref, s2_ref):
    b = pl.program_id(0)
    xnb = xn_ref[0].astype(jnp.bfloat16)
    mdb = md_ref[0].astype(jnp.bfloat16)
    y = lax.dot_general(we_ref[...], xnb, (((1,), (1,)), ((), ())),
                        preferred_element_type=jnp.float32)
    y = y + lax.dot_general(wo_ref[...], mdb, (((1,), (1,)), ((), ())),
                            preferred_element_type=jnp.float32)
    y = y + bias_ref[...]
    y_ref[0] = y
    ps1 = jnp.sum(y, axis=1, keepdims=True)
    ps2 = jnp.sum(y * y, axis=1, keepdims=True)

    @pl.when(b == 0)
    def _():
        s1_ref[...] = ps1
        s2_ref[...] = ps2

    @pl.when(b != 0)
    def _():
        s1_ref[...] += ps1
        s2_ref[...] += ps2


def _bn_gelu_kernel(y_ref, s1_ref, s2_ref, gamma_ref, beta_ref, o_ref,
                    *, count):
    mean = s1_ref[...] * (1.0 / count)
    var = s2_ref[...] * (1.0 / count) - mean * mean
    scale = gamma_ref[...] * lax.rsqrt(var + 1e-5)
    shift = beta_ref[...] - mean * scale
    yn = y_ref[0] * scale + shift
    o_ref[0] = yn * 0.5 * (1.0 + lax.erf(yn * 0.7071067811865476))


def kernel(x, W, b, gamma, beta):
    B, C, N = x.shape
    O = W.shape[0]
    we = W[:, 0::2].astype(jnp.bfloat16)
    wo = W[:, 1::2].astype(jnp.bfloat16)

    xnt, nn = pl.pallas_call(
        _prep_kernel,
        grid=(B,),
        in_specs=[pl.BlockSpec((1, C, N), lambda i: (i, 0, 0))],
        out_specs=[pl.BlockSpec((1, N, C), lambda i: (i, 0, 0)),
                   pl.BlockSpec((1, N, K_NB), lambda i: (i, 0, 0))],
        out_shape=[jax.ShapeDtypeStruct((B, N, C), jnp.float32),
                   jax.ShapeDtypeStruct((B, N, K_NB), jnp.int32)],
    )(x)

    md = pl.kernel(
        functools.partial(_knn_sc_kernel, n=N, c=C),
        mesh=plsc.VectorSubcoreMesh(core_axis_name="c", subcore_axis_name="s"),
        compiler_params=pltpu.CompilerParams(use_tc_tiling_on_sc=False),
        out_type=jax.ShapeDtypeStruct((B, N, C), jnp.float32),
        scratch_types=[
            pltpu.VMEM((N, C), jnp.float32),
            pltpu.VMEM((N, K_NB), jnp.int32),
            pltpu.VMEM((_CHK, C), jnp.float32),
        ],
    )(xnt, nn)

    y, s1, s2 = pl.pallas_call(
        _conv_kernel,
        grid=(B,),
        in_specs=[pl.BlockSpec((O, C), lambda i: (0, 0)),
                  pl.BlockSpec((O, C), lambda i: (0, 0)),
                  pl.BlockSpec((O, 1), lambda i: (0, 0)),
                  pl.BlockSpec((1, N, C), lambda i: (i, 0, 0)),
                  pl.BlockSpec((1, N, C), lambda i: (i, 0, 0))],
        out_specs=[pl.BlockSpec((1, O, N), lambda i: (i, 0, 0)),
                   pl.BlockSpec((O, 1), lambda i: (0, 0)),
                   pl.BlockSpec((O, 1), lambda i: (0, 0))],
        out_shape=[jax.ShapeDtypeStruct((B, O, N), jnp.float32),
                   jax.ShapeDtypeStruct((O, 1), jnp.float32),
                   jax.ShapeDtypeStruct((O, 1), jnp.float32)],
    )(we, wo, b.reshape(O, 1), xnt, md)

    out = pl.pallas_call(
        functools.partial(_bn_gelu_kernel, count=float(B * N)),
        grid=(B,),
        in_specs=[pl.BlockSpec((1, O, N), lambda i: (i, 0, 0)),
                  pl.BlockSpec((O, 1), lambda i: (0, 0)),
                  pl.BlockSpec((O, 1), lambda i: (0, 0)),
                  pl.BlockSpec((O, 1), lambda i: (0, 0)),
                  pl.BlockSpec((O, 1), lambda i: (0, 0))],
        out_specs=pl.BlockSpec((1, O, N), lambda i: (i, 0, 0)),
        out_shape=jax.ShapeDtypeStruct((B, O, N), jnp.float32),
    )(y, s1, s2, gamma.reshape(O, 1), beta.reshape(O, 1))

    return out.reshape(B, O, N, 1)


# speedup vs baseline: 1.9799x; 1.2332x over previous
import functools

import jax
import jax.numpy as jnp
from jax import lax
from jax.experimental import pallas as pl
from jax.experimental.pallas import tpu as pltpu
from jax.experimental.pallas import tpu_sc as plsc

K_NB = 16
_BIG = 1e9
_L = 16
_CHK = 64
_PAD = 128


def _prep_kernel(x_ref, xe_ref):
    x = x_ref[0]
    xt = jnp.transpose(x)
    nrm = jnp.sqrt(jnp.sum(xt * xt, axis=1, keepdims=True))
    xn = xt * (1.0 / jnp.maximum(nrm, 1e-12))
    sq = jnp.sum(xn * xn, axis=1, keepdims=True)
    g = lax.dot_general(xn, xn, (((1,), (1,)), ((), ())),
                        preferred_element_type=jnp.float32)
    d2 = sq + jnp.transpose(sq) - 2.0 * g
    d2 = jnp.maximum(d2, 0.0)
    N = d2.shape[0]
    rowid = lax.broadcasted_iota(jnp.int32, (N, N), 0)
    colid = lax.broadcasted_iota(jnp.int32, (N, N), 1)
    d2 = jnp.where(rowid == colid, _BIG, d2)
    rows = []
    for _ in range(K_NB):
        m = jnp.min(d2, axis=0, keepdims=True)
        cand = jnp.where(d2 == m, rowid, N)
        first = jnp.min(cand, axis=0, keepdims=True)
        rows.append(first)
        d2 = jnp.where(rowid == first, _BIG, d2)
    nnf = jnp.transpose(
        jnp.concatenate(rows, axis=0).astype(jnp.float32))
    pad = jnp.zeros((N, _PAD - K_NB), jnp.float32)
    xe_ref[0] = jnp.concatenate([xn, nnf, pad], axis=1)


def _knn_sc_kernel(xe_hbm, md_hbm, xnt_v, idx_v, md_v, *, n, c):
    nc = 2
    b = lax.axis_index("s") * nc + lax.axis_index("c")
    pltpu.sync_copy(xe_hbm.at[b, :, pl.ds(0, c)], xnt_v)
    pltpu.sync_copy(xe_hbm.at[b, :, pl.ds(c, K_NB)], idx_v)
    ncc = c // _L
    nh = ncc // 2

    def chunk_body(ch, carry0):
        def row_body(il, carry1):
            i = ch * _CHK + il
            idxv = idx_v[i, :].astype(jnp.int32)
            for h in range(2):
                lo = h * nh
                mx = [xnt_v[i, pl.ds((lo + cc) * _L, _L)] for cc in range(nh)]
                mn = list(mx)
                for t in range(K_NB):
                    jsc = idxv[t]
                    for cc in range(nh):
                        nb = xnt_v[jsc, pl.ds((lo + cc) * _L, _L)]
                        mx[cc] = jnp.maximum(mx[cc], nb)
                        mn[cc] = jnp.minimum(mn[cc], nb)
                for cc in range(nh):
                    xi = xnt_v[i, pl.ds((lo + cc) * _L, _L)]
                    md_v[il, pl.ds((lo + cc) * _L, _L)] = jnp.maximum(
                        mx[cc] - xi, xi - mn[cc])
            return carry1

        lax.fori_loop(0, _CHK, row_body, 0)
        pltpu.sync_copy(md_v, md_hbm.at[b, pl.ds(ch * _CHK, _CHK)])
        return carry0

    lax.fori_loop(0, n // _CHK, chunk_body, 0)


def _conv_kernel(we_ref, wo_ref, bias_ref, xe_ref, md_ref,
                 y_ref, s1_ref, s2_ref, *, c):
    b = pl.program_id(0)
    xnb = xe_ref[0][:, :c].astype(jnp.bfloat16)
    mdb = md_ref[0].astype(jnp.bfloat16)
    y = lax.dot_general(we_ref[...], xnb, (((1,), (1,)), ((), ())),
                        preferred_element_type=jnp.float32)
    y = y + lax.dot_general(wo_ref[...], mdb, (((1,), (1,)), ((), ())),
                            preferred_element_type=jnp.float32)
    y = y + bias_ref[...]
    y_ref[0] = y
    ps1 = jnp.sum(y, axis=1, keepdims=True)
    ps2 = jnp.sum(y * y, axis=1, keepdims=True)

    @pl.when(b == 0)
    def _():
        s1_ref[...] = ps1
        s2_ref[...] = ps2

    @pl.when(b != 0)
    def _():
        s1_ref[...] += ps1
        s2_ref[...] += ps2


def _bn_gelu_kernel(y_ref, s1_ref, s2_ref, gamma_ref, beta_ref, o_ref,
                    *, count):
    mean = s1_ref[...] * (1.0 / count)
    var = s2_ref[...] * (1.0 / count) - mean * mean
    scale = gamma_ref[...] * lax.rsqrt(var + 1e-5)
    shift = beta_ref[...] - mean * scale
    yn = y_ref[0] * scale + shift
    o_ref[0] = yn * 0.5 * (1.0 + lax.erf(yn * 0.7071067811865476))


def kernel(x, W, b, gamma, beta):
    B, C, N = x.shape
    O = W.shape[0]
    CE = C + _PAD
    we = W[:, 0::2].astype(jnp.bfloat16)
    wo = W[:, 1::2].astype(jnp.bfloat16)

    xe = pl.pallas_call(
        _prep_kernel,
        grid=(B,),
        in_specs=[pl.BlockSpec((1, C, N), lambda i: (i, 0, 0))],
        out_specs=pl.BlockSpec((1, N, CE), lambda i: (i, 0, 0)),
        out_shape=jax.ShapeDtypeStruct((B, N, CE), jnp.float32),
    )(x)

    md = pl.kernel(
        functools.partial(_knn_sc_kernel, n=N, c=C),
        mesh=plsc.VectorSubcoreMesh(core_axis_name="c", subcore_axis_name="s"),
        compiler_params=pltpu.CompilerParams(use_tc_tiling_on_sc=False),
        out_type=jax.ShapeDtypeStruct((B, N, C), jnp.float32),
        scratch_types=[
            pltpu.VMEM((N, C), jnp.float32),
            pltpu.VMEM((N, K_NB), jnp.float32),
            pltpu.VMEM((_CHK, C), jnp.float32),
        ],
    )(xe)

    y, s1, s2 = pl.pallas_call(
        functools.partial(_conv_kernel, c=C),
        grid=(B,),
        in_specs=[pl.BlockSpec((O, C), lambda i: (0, 0)),
                  pl.BlockSpec((O, C), lambda i: (0, 0)),
                  pl.BlockSpec((O, 1), lambda i: (0, 0)),
                  pl.BlockSpec((1, N, CE), lambda i: (i, 0, 0)),
                  pl.BlockSpec((1, N, C), lambda i: (i, 0, 0))],
        out_specs=[pl.BlockSpec((1, O, N), lambda i: (i, 0, 0)),
                   pl.BlockSpec((O, 1), lambda i: (0, 0)),
                   pl.BlockSpec((O, 1), lambda i: (0, 0))],
        out_shape=[jax.ShapeDtypeStruct((B, O, N), jnp.float32),
                   jax.ShapeDtypeStruct((O, 1), jnp.float32),
                   jax.ShapeDtypeStruct((O, 1), jnp.float32)],
    )(we, wo, b.reshape(O, 1), xe, md)

    out = pl.pallas_call(
        functools.partial(_bn_gelu_kernel, count=float(B * N)),
        grid=(B,),
        in_specs=[pl.BlockSpec((1, O, N), lambda i: (i, 0, 0)),
                  pl.BlockSpec((O, 1), lambda i: (0, 0)),
                  pl.BlockSpec((O, 1), lambda i: (0, 0)),
                  pl.BlockSpec((O, 1), lambda i: (0, 0)),
                  pl.BlockSpec((O, 1), lambda i: (0, 0))],
        out_specs=pl.BlockSpec((1, O, N), lambda i: (i, 0, 0)),
        out_shape=jax.ShapeDtypeStruct((B, O, N), jnp.float32),
    )(y, s1, s2, gamma.reshape(O, 1), beta.reshape(O, 1))

    return out.reshape(B, O, N, 1)
